# Initial kernel scaffold; baseline (speedup 1.0000x reference)
import jax
import jax.numpy as jnp
from jax.experimental import pallas as pl


def kernel(x, edge_index, W1, b1, W2, b2):
    raise NotImplementedError("write your pallas kernel here")


# speedup vs baseline: 19.7241x; 19.7241x over previous
import functools

import jax
import jax.numpy as jnp
from jax import lax
from jax.experimental import pallas as pl
from jax.experimental.pallas import tpu as pltpu
from jax.experimental.pallas import tpu_sc as plsc

N = 10000
NFEAT = 128
NHID = 16
NPAD = 10240
E = 320000
NCORES = 2
NSUB = 16
NWORK = NCORES * NSUB
CHUNK = 128
EPW = 10240
NCHUNKS = EPW // CHUNK
EPAD = NWORK * EPW
RPS = NPAD // NSUB

_mesh = plsc.VectorSubcoreMesh(core_axis_name="c", subcore_axis_name="s")
_sc_params = pltpu.CompilerParams(use_tc_tiling_on_sc=False)


def _zero_fill(buf, nrows):
    @pl.loop(0, nrows)
    def _(i):
        buf[i, :] = jnp.zeros((16,), jnp.float32)


@functools.partial(
    pl.kernel,
    out_type=jax.ShapeDtypeStruct((NCORES * NPAD, 16), jnp.float32),
    mesh=_mesh,
    scratch_types=[
        pltpu.VMEM((CHUNK, 16), jnp.float32),
        pltpu.VMEM((1, CHUNK), jnp.int32),
        pltpu.VMEM((RPS, 16), jnp.float32),
        pltpu.VMEM_SHARED((NPAD, 16), jnp.float32),
    ],
    compiler_params=_sc_params,
)
def _deg_pass(ei_hbm, out_hbm, ones_v, idx_v, zero_v, accum):
    cid = lax.axis_index("c")
    sid = lax.axis_index("s")

    @pl.loop(0, CHUNK)
    def _(i):
        ones_v[i, :] = jnp.ones((16,), jnp.float32)

    _zero_fill(zero_v, RPS)
    pltpu.sync_copy(zero_v, accum.at[pl.ds(sid * RPS, RPS)])
    plsc.subcore_barrier()

    base = (cid * NSUB + sid) * EPW

    @pl.loop(0, NCHUNKS)
    def _(k):
        pltpu.sync_copy(ei_hbm.at[pl.ds(0, 1), pl.ds(base + k * CHUNK, CHUNK)],
                        idx_v)
        pltpu.sync_copy(ones_v, accum.at[idx_v.at[0]], add=True)

    plsc.subcore_barrier()
    pltpu.sync_copy(
        accum.at[pl.ds(sid * RPS, RPS)],
        out_hbm.at[pl.ds(cid * NPAD + sid * RPS, RPS)],
    )


@functools.partial(
    pl.kernel,
    out_type=jax.ShapeDtypeStruct((NCORES * NPAD, 16), jnp.float32),
    mesh=_mesh,
    scratch_types=[
        pltpu.VMEM((1, CHUNK), jnp.int32),
        pltpu.VMEM((1, CHUNK), jnp.int32),
        pltpu.VMEM((CHUNK, 16), jnp.float32),
        pltpu.VMEM((RPS, 16), jnp.float32),
        pltpu.VMEM_SHARED((NPAD, 16), jnp.float32),
    ],
    compiler_params=_sc_params,
)
def _agg_pass(hp_hbm, ei_hbm, out_hbm, rv, cv, msg_v, zero_v, accum):
    cid = lax.axis_index("c")
    sid = lax.axis_index("s")

    _zero_fill(zero_v, RPS)
    pltpu.sync_copy(zero_v, accum.at[pl.ds(sid * RPS, RPS)])
    plsc.subcore_barrier()

    base = (cid * NSUB + sid) * EPW

    @pl.loop(0, NCHUNKS)
    def _(k):
        off = base + k * CHUNK
        pltpu.sync_copy(ei_hbm.at[pl.ds(0, 1), pl.ds(off, CHUNK)], rv)
        pltpu.sync_copy(ei_hbm.at[pl.ds(1, 1), pl.ds(off, CHUNK)], cv)
        pltpu.sync_copy(hp_hbm.at[rv.at[0]], msg_v)
        pltpu.sync_copy(msg_v, accum.at[cv.at[0]], add=True)

    plsc.subcore_barrier()
    pltpu.sync_copy(
        accum.at[pl.ds(sid * RPS, RPS)],
        out_hbm.at[pl.ds(cid * NPAD + sid * RPS, RPS)],
    )


def _tc_pre(deg_ref, x_ref, w1t_ref, b1_ref, hp_ref, dis_ref):
    deg = deg_ref[0, :N, :] + deg_ref[1, :N, :] + 1.0
    dis = lax.rsqrt(deg)
    h1 = lax.dot_general(
        x_ref[...], w1t_ref[...], (((1,), (0,)), ((), ())),
        preferred_element_type=jnp.float32,
    ) + b1_ref[...]
    hp_ref[:N, :] = dis * h1
    hp_ref[N:, :] = jnp.zeros((NPAD - N, 16), jnp.float32)
    dis_ref[...] = dis


def _tc_mid(p_ref, hp1_ref, dis_ref, w2t_ref, b2_ref, hp2_ref):
    dis = dis_ref[...]
    s = p_ref[0, :N, :] + p_ref[1, :N, :] + hp1_ref[:N, :]
    out1 = jnp.maximum(dis * s, 0.0)
    h2 = lax.dot_general(
        out1, w2t_ref[...], (((1,), (0,)), ((), ())),
        preferred_element_type=jnp.float32,
    ) + b2_ref[...]
    hp2_ref[:N, :] = dis * h2
    hp2_ref[N:, :] = jnp.zeros((NPAD - N, 16), jnp.float32)


def _tc_post(q_ref, hp2_ref, dis_ref, o_ref):
    o = dis_ref[...] * (q_ref[0, :N, :] + q_ref[1, :N, :] + hp2_ref[:N, :])
    m = jnp.max(o, axis=1, keepdims=True)
    lse = jnp.log(jnp.sum(jnp.exp(o - m), axis=1, keepdims=True)) + m
    o_ref[...] = o - lse


_pre_call = pl.pallas_call(
    _tc_pre,
    out_shape=(
        jax.ShapeDtypeStruct((NPAD, 16), jnp.float32),
        jax.ShapeDtypeStruct((N, 16), jnp.float32),
    ),
)

_mid_call = pl.pallas_call(
    _tc_mid,
    out_shape=jax.ShapeDtypeStruct((NPAD, 16), jnp.float32),
)

_post_call = pl.pallas_call(
    _tc_post,
    out_shape=jax.ShapeDtypeStruct((N, 16), jnp.float32),
)


def kernel(x, edge_index, W1, b1, W2, b2):
    pad = jnp.full((2, EPAD - E), N, jnp.int32)
    ei = jnp.concatenate([edge_index, pad], axis=1)

    degp = _deg_pass(ei).reshape(NCORES, NPAD, 16)
    hp1, dis = _pre_call(degp, x, W1.T, b1.reshape(1, NHID))
    p = _agg_pass(hp1, ei).reshape(NCORES, NPAD, 16)
    hp2 = _mid_call(p, hp1, dis, W2.T, b2.reshape(1, NHID))
    q = _agg_pass(hp2, ei).reshape(NCORES, NPAD, 16)
    return _post_call(q, hp2, dis)


# speedup vs baseline: 53.0032x; 2.6872x over previous
import functools

import jax
import jax.numpy as jnp
from jax import lax
from jax.experimental import pallas as pl
from jax.experimental.pallas import tpu as pltpu
from jax.experimental.pallas import tpu_sc as plsc

N = 10000
NFEAT = 128
NHID = 16
NPAD = 10240
E = 320000
NCORES = 2
NSUB = 16
NWORK = NCORES * NSUB
CHUNK = 128
EPW = 10240
NCHUNKS = EPW // CHUNK
EPAD = NWORK * EPW
RPS = NPAD // NSUB

_mesh = plsc.VectorSubcoreMesh(core_axis_name="c", subcore_axis_name="s")
_sc_params = pltpu.CompilerParams(use_tc_tiling_on_sc=False)


def _zero_fill(buf, nrows):
    @pl.loop(0, nrows)
    def _(i):
        buf[i, :] = jnp.zeros((16,), jnp.float32)


@functools.partial(
    pl.kernel,
    out_type=jax.ShapeDtypeStruct((NCORES * NPAD, 16), jnp.float32),
    mesh=_mesh,
    scratch_types=[
        pltpu.VMEM((CHUNK, 16), jnp.float32),
        pltpu.VMEM((NCHUNKS, CHUNK), jnp.int32),
        pltpu.VMEM((RPS, 16), jnp.float32),
        pltpu.VMEM_SHARED((NPAD, 16), jnp.float32),
    ],
    compiler_params=_sc_params,
)
def _deg_pass(row_hbm, out_hbm, ones_v, rv_all, zero_v, accum):
    cid = lax.axis_index("c")
    sid = lax.axis_index("s")

    @pl.loop(0, CHUNK)
    def _(i):
        ones_v[i, :] = jnp.ones((16,), jnp.float32)

    _zero_fill(zero_v, RPS)
    pltpu.sync_copy(zero_v, accum.at[pl.ds(sid * RPS, RPS)])
    wbase = (cid * NSUB + sid) * NCHUNKS
    pltpu.sync_copy(row_hbm.at[pl.ds(wbase, NCHUNKS)], rv_all)
    plsc.subcore_barrier()

    @pl.loop(0, NCHUNKS)
    def _(k):
        pltpu.sync_copy(ones_v, accum.at[rv_all.at[k]], add=True)

    plsc.subcore_barrier()
    pltpu.sync_copy(
        accum.at[pl.ds(sid * RPS, RPS)],
        out_hbm.at[pl.ds(cid * NPAD + sid * RPS, RPS)],
    )


@functools.partial(
    pl.kernel,
    out_type=jax.ShapeDtypeStruct((NCORES * NPAD, 16), jnp.float32),
    mesh=_mesh,
    scratch_types=[
        pltpu.VMEM((NCHUNKS, CHUNK), jnp.int32),
        pltpu.VMEM((NCHUNKS, CHUNK), jnp.int32),
        pltpu.VMEM((CHUNK, 16), jnp.float32),
        pltpu.VMEM((RPS, 16), jnp.float32),
        pltpu.VMEM_SHARED((NPAD, 16), jnp.float32),
        pltpu.VMEM_SHARED((NPAD, 16), jnp.float32),
    ],
    compiler_params=_sc_params,
)
def _agg_pass(hp_hbm, row_hbm, col_hbm, out_hbm, rv_all, cv_all, msg_v,
              zero_v, hp_s, accum):
    cid = lax.axis_index("c")
    sid = lax.axis_index("s")

    _zero_fill(zero_v, RPS)
    pltpu.sync_copy(zero_v, accum.at[pl.ds(sid * RPS, RPS)])
    pltpu.sync_copy(hp_hbm.at[pl.ds(sid * RPS, RPS)],
                    hp_s.at[pl.ds(sid * RPS, RPS)])
    wbase = (cid * NSUB + sid) * NCHUNKS
    pltpu.sync_copy(row_hbm.at[pl.ds(wbase, NCHUNKS)], rv_all)
    pltpu.sync_copy(col_hbm.at[pl.ds(wbase, NCHUNKS)], cv_all)
    plsc.subcore_barrier()

    @pl.loop(0, NCHUNKS)
    def _(k):
        pltpu.sync_copy(hp_s.at[rv_all.at[k]], msg_v)
        pltpu.sync_copy(msg_v, accum.at[cv_all.at[k]], add=True)

    plsc.subcore_barrier()
    pltpu.sync_copy(
        accum.at[pl.ds(sid * RPS, RPS)],
        out_hbm.at[pl.ds(cid * NPAD + sid * RPS, RPS)],
    )


def _tc_pre(deg_ref, x_ref, w1t_ref, b1_ref, hp_ref, dis_ref):
    deg = deg_ref[0, :N, :] + deg_ref[1, :N, :] + 1.0
    dis = lax.rsqrt(deg)
    h1 = lax.dot_general(
        x_ref[...], w1t_ref[...], (((1,), (0,)), ((), ())),
        preferred_element_type=jnp.float32,
    ) + b1_ref[...]
    hp_ref[:N, :] = dis * h1
    hp_ref[N:, :] = jnp.zeros((NPAD - N, 16), jnp.float32)
    dis_ref[...] = dis


def _tc_mid(p_ref, hp1_ref, dis_ref, w2t_ref, b2_ref, hp2_ref):
    dis = dis_ref[...]
    s = p_ref[0, :N, :] + p_ref[1, :N, :] + hp1_ref[:N, :]
    out1 = jnp.maximum(dis * s, 0.0)
    h2 = lax.dot_general(
        out1, w2t_ref[...], (((1,), (0,)), ((), ())),
        preferred_element_type=jnp.float32,
    ) + b2_ref[...]
    hp2_ref[:N, :] = dis * h2
    hp2_ref[N:, :] = jnp.zeros((NPAD - N, 16), jnp.float32)


def _tc_post(q_ref, hp2_ref, dis_ref, o_ref):
    o = dis_ref[...] * (q_ref[0, :N, :] + q_ref[1, :N, :] + hp2_ref[:N, :])
    m = jnp.max(o, axis=1, keepdims=True)
    lse = jnp.log(jnp.sum(jnp.exp(o - m), axis=1, keepdims=True)) + m
    o_ref[...] = o - lse


_pre_call = pl.pallas_call(
    _tc_pre,
    out_shape=(
        jax.ShapeDtypeStruct((NPAD, 16), jnp.float32),
        jax.ShapeDtypeStruct((N, 16), jnp.float32),
    ),
)

_mid_call = pl.pallas_call(
    _tc_mid,
    out_shape=jax.ShapeDtypeStruct((NPAD, 16), jnp.float32),
)

_post_call = pl.pallas_call(
    _tc_post,
    out_shape=jax.ShapeDtypeStruct((N, 16), jnp.float32),
)


def kernel(x, edge_index, W1, b1, W2, b2):
    pad = jnp.full((2, EPAD - E), N, jnp.int32)
    ei = jnp.concatenate([edge_index, pad], axis=1)
    row2d = ei[0].reshape(NWORK * NCHUNKS, CHUNK)
    col2d = ei[1].reshape(NWORK * NCHUNKS, CHUNK)

    degp = _deg_pass(row2d).reshape(NCORES, NPAD, 16)
    hp1, dis = _pre_call(degp, x, W1.T, b1.reshape(1, NHID))
    p = _agg_pass(hp1, row2d, col2d).reshape(NCORES, NPAD, 16)
    hp2 = _mid_call(p, hp1, dis, W2.T, b2.reshape(1, NHID))
    q = _agg_pass(hp2, row2d, col2d).reshape(NCORES, NPAD, 16)
    return _post_call(q, hp2, dis)


# speedup vs baseline: 54.7682x; 1.0333x over previous
import functools

import jax
import jax.numpy as jnp
from jax import lax
from jax.experimental import pallas as pl
from jax.experimental.pallas import tpu as pltpu
from jax.experimental.pallas import tpu_sc as plsc

N = 10000
NFEAT = 128
NHID = 16
NPAD = 10240
E = 320000
NCORES = 2
NSUB = 16
NWORK = NCORES * NSUB
CHUNK = 128
EPW = 10240
NCHUNKS = EPW // CHUNK
EPAD = NWORK * EPW
RPS = NPAD // NSUB
NBUF = 4

_mesh = plsc.VectorSubcoreMesh(core_axis_name="c", subcore_axis_name="s")
_sc_params = pltpu.CompilerParams(use_tc_tiling_on_sc=False)


def _zero_fill(buf, nrows):
    @pl.loop(0, nrows)
    def _(i):
        buf[i, :] = jnp.zeros((16,), jnp.float32)


@functools.partial(
    pl.kernel,
    out_type=jax.ShapeDtypeStruct((NCORES * NPAD, 16), jnp.float32),
    mesh=_mesh,
    scratch_types=[
        pltpu.VMEM((CHUNK, 16), jnp.float32),
        pltpu.VMEM((NCHUNKS, CHUNK), jnp.int32),
        pltpu.VMEM((RPS, 16), jnp.float32),
        pltpu.VMEM_SHARED((NPAD, 16), jnp.float32),
        pltpu.SemaphoreType.DMA,
    ],
    compiler_params=_sc_params,
)
def _deg_pass(row_hbm, out_hbm, ones_v, rv_all, zero_v, accum, sem):
    cid = lax.axis_index("c")
    sid = lax.axis_index("s")

    @pl.loop(0, CHUNK)
    def _(i):
        ones_v[i, :] = jnp.ones((16,), jnp.float32)

    _zero_fill(zero_v, RPS)
    pltpu.sync_copy(zero_v, accum.at[pl.ds(sid * RPS, RPS)])
    wbase = (cid * NSUB + sid) * NCHUNKS
    pltpu.sync_copy(row_hbm.at[pl.ds(wbase, NCHUNKS)], rv_all)
    plsc.subcore_barrier()

    @pl.loop(0, NCHUNKS)
    def _(k):
        pltpu.async_copy(ones_v, accum.at[rv_all.at[k]], sem, add=True)

    @pl.loop(0, NCHUNKS)
    def _(k):
        pltpu.make_async_copy(ones_v, accum.at[rv_all.at[0]], sem).wait()

    plsc.subcore_barrier()
    pltpu.sync_copy(
        accum.at[pl.ds(sid * RPS, RPS)],
        out_hbm.at[pl.ds(cid * NPAD + sid * RPS, RPS)],
    )


@functools.partial(
    pl.kernel,
    out_type=jax.ShapeDtypeStruct((NCORES * NPAD, 16), jnp.float32),
    mesh=_mesh,
    scratch_types=[
        pltpu.VMEM((NCHUNKS, CHUNK), jnp.int32),
        pltpu.VMEM((NCHUNKS, CHUNK), jnp.int32),
        pltpu.VMEM((NBUF, CHUNK, 16), jnp.float32),
        pltpu.VMEM((RPS, 16), jnp.float32),
        pltpu.VMEM_SHARED((NPAD, 16), jnp.float32),
        pltpu.VMEM_SHARED((NPAD, 16), jnp.float32),
        pltpu.SemaphoreType.DMA,
        pltpu.SemaphoreType.DMA,
    ],
    compiler_params=_sc_params,
)
def _agg_pass(hp_hbm, row_hbm, col_hbm, out_hbm, rv_all, cv_all, msg_v,
              zero_v, hp_s, accum, sem_g, sem_s):
    cid = lax.axis_index("c")
    sid = lax.axis_index("s")

    _zero_fill(zero_v, RPS)
    pltpu.sync_copy(zero_v, accum.at[pl.ds(sid * RPS, RPS)])
    pltpu.sync_copy(hp_hbm.at[pl.ds(sid * RPS, RPS)],
                    hp_s.at[pl.ds(sid * RPS, RPS)])
    wbase = (cid * NSUB + sid) * NCHUNKS
    pltpu.sync_copy(row_hbm.at[pl.ds(wbase, NCHUNKS)], rv_all)
    pltpu.sync_copy(col_hbm.at[pl.ds(wbase, NCHUNKS)], cv_all)
    plsc.subcore_barrier()

    @pl.loop(0, NCHUNKS, step=NBUF)
    def _(k):
        for j in range(NBUF):
            pltpu.async_copy(hp_s.at[rv_all.at[k + j]], msg_v.at[j], sem_g)
        for j in range(NBUF):
            pltpu.make_async_copy(hp_s.at[rv_all.at[k]], msg_v.at[0],
                                  sem_g).wait()
        for j in range(NBUF):
            pltpu.async_copy(msg_v.at[j], accum.at[cv_all.at[k + j]], sem_s,
                             add=True)
        for j in range(NBUF):
            pltpu.make_async_copy(msg_v.at[0], accum.at[cv_all.at[k]],
                                  sem_s).wait()

    plsc.subcore_barrier()
    pltpu.sync_copy(
        accum.at[pl.ds(sid * RPS, RPS)],
        out_hbm.at[pl.ds(cid * NPAD + sid * RPS, RPS)],
    )


def _tc_pre(deg_ref, x_ref, w1t_ref, b1_ref, hp_ref, dis_ref):
    deg = deg_ref[0, :N, :] + deg_ref[1, :N, :] + 1.0
    dis = lax.rsqrt(deg)
    h1 = lax.dot_general(
        x_ref[...], w1t_ref[...], (((1,), (0,)), ((), ())),
        preferred_element_type=jnp.float32,
    ) + b1_ref[...]
    hp_ref[:N, :] = dis * h1
    hp_ref[N:, :] = jnp.zeros((NPAD - N, 16), jnp.float32)
    dis_ref[...] = dis


def _tc_mid(p_ref, hp1_ref, dis_ref, w2t_ref, b2_ref, hp2_ref):
    dis = dis_ref[...]
    s = p_ref[0, :N, :] + p_ref[1, :N, :] + hp1_ref[:N, :]
    out1 = jnp.maximum(dis * s, 0.0)
    h2 = lax.dot_general(
        out1, w2t_ref[...], (((1,), (0,)), ((), ())),
        preferred_element_type=jnp.float32,
    ) + b2_ref[...]
    hp2_ref[:N, :] = dis * h2
    hp2_ref[N:, :] = jnp.zeros((NPAD - N, 16), jnp.float32)


def _tc_post(q_ref, hp2_ref, dis_ref, o_ref):
    o = dis_ref[...] * (q_ref[0, :N, :] + q_ref[1, :N, :] + hp2_ref[:N, :])
    m = jnp.max(o, axis=1, keepdims=True)
    lse = jnp.log(jnp.sum(jnp.exp(o - m), axis=1, keepdims=True)) + m
    o_ref[...] = o - lse


_pre_call = pl.pallas_call(
    _tc_pre,
    out_shape=(
        jax.ShapeDtypeStruct((NPAD, 16), jnp.float32),
        jax.ShapeDtypeStruct((N, 16), jnp.float32),
    ),
)

_mid_call = pl.pallas_call(
    _tc_mid,
    out_shape=jax.ShapeDtypeStruct((NPAD, 16), jnp.float32),
)

_post_call = pl.pallas_call(
    _tc_post,
    out_shape=jax.ShapeDtypeStruct((N, 16), jnp.float32),
)


def kernel(x, edge_index, W1, b1, W2, b2):
    pad = jnp.full((2, EPAD - E), N, jnp.int32)
    ei = jnp.concatenate([edge_index, pad], axis=1)
    row2d = ei[0].reshape(NWORK * NCHUNKS, CHUNK)
    col2d = ei[1].reshape(NWORK * NCHUNKS, CHUNK)

    degp = _deg_pass(row2d).reshape(NCORES, NPAD, 16)
    hp1, dis = _pre_call(degp, x, W1.T, b1.reshape(1, NHID))
    p = _agg_pass(hp1, row2d, col2d).reshape(NCORES, NPAD, 16)
    hp2 = _mid_call(p, hp1, dis, W2.T, b2.reshape(1, NHID))
    q = _agg_pass(hp2, row2d, col2d).reshape(NCORES, NPAD, 16)
    return _post_call(q, hp2, dis)


# speedup vs baseline: 54.8555x; 1.0016x over previous
import functools

import jax
import jax.numpy as jnp
from jax import lax
from jax.experimental import pallas as pl
from jax.experimental.pallas import tpu as pltpu
from jax.experimental.pallas import tpu_sc as plsc

N = 10000
NFEAT = 128
NHID = 16
NPAD = 10240
E = 320000
NCORES = 2
NSUB = 16
NWORK = NCORES * NSUB
CHUNK = 128
EPW = 10240
NCHUNKS = EPW // CHUNK
EPAD = NWORK * EPW
RPS = NPAD // NSUB
NBUF = 4

_mesh = plsc.VectorSubcoreMesh(core_axis_name="c", subcore_axis_name="s")
_sc_params = pltpu.CompilerParams(use_tc_tiling_on_sc=False)


def _zero_fill(buf, nrows):
    @pl.loop(0, nrows)
    def _(i):
        buf[i, :] = jnp.zeros((16,), jnp.float32)


@functools.partial(
    pl.kernel,
    out_type=jax.ShapeDtypeStruct((NCORES * NPAD, 16), jnp.float32),
    mesh=_mesh,
    scratch_types=[
        pltpu.VMEM((CHUNK, 16), jnp.float32),
        pltpu.VMEM((NCHUNKS, CHUNK), jnp.int32),
        pltpu.VMEM((RPS, 16), jnp.float32),
        pltpu.VMEM_SHARED((NPAD, 16), jnp.float32),
        pltpu.SemaphoreType.DMA,
    ],
    compiler_params=_sc_params,
)
def _deg_pass(row_hbm, out_hbm, ones_v, rv_all, zero_v, accum, sem):
    cid = lax.axis_index("c")
    sid = lax.axis_index("s")

    @pl.loop(0, CHUNK)
    def _(i):
        ones_v[i, :] = jnp.ones((16,), jnp.float32)

    _zero_fill(zero_v, RPS)
    pltpu.sync_copy(zero_v, accum.at[pl.ds(sid * RPS, RPS)])
    wbase = (cid * NSUB + sid) * NCHUNKS
    pltpu.sync_copy(row_hbm.at[pl.ds(wbase, NCHUNKS)], rv_all)
    plsc.subcore_barrier()

    @pl.loop(0, NCHUNKS)
    def _(k):
        pltpu.async_copy(ones_v, accum.at[rv_all.at[k]], sem, add=True)

    @pl.loop(0, NCHUNKS)
    def _(k):
        pltpu.make_async_copy(ones_v, accum.at[rv_all.at[0]], sem).wait()

    plsc.subcore_barrier()
    pltpu.sync_copy(
        accum.at[pl.ds(sid * RPS, RPS)],
        out_hbm.at[pl.ds(cid * NPAD + sid * RPS, RPS)],
    )


@functools.partial(
    pl.kernel,
    out_type=jax.ShapeDtypeStruct((NCORES * NPAD, 16), jnp.float32),
    mesh=_mesh,
    scratch_types=[
        pltpu.VMEM((NCHUNKS, CHUNK), jnp.int32),
        pltpu.VMEM((NCHUNKS, CHUNK), jnp.int32),
        pltpu.VMEM((NBUF, CHUNK, 16), jnp.float32),
        pltpu.VMEM((RPS, 16), jnp.float32),
        pltpu.VMEM_SHARED((NPAD, 16), jnp.float32),
        pltpu.VMEM_SHARED((NPAD, 16), jnp.float32),
        pltpu.SemaphoreType.DMA,
        pltpu.SemaphoreType.DMA,
    ],
    compiler_params=_sc_params,
)
def _agg_pass(hp_hbm, row_hbm, col_hbm, out_hbm, rv_all, cv_all, msg_v,
              zero_v, hp_s, accum, sem_g, sem_s):
    cid = lax.axis_index("c")
    sid = lax.axis_index("s")

    _zero_fill(zero_v, RPS)
    pltpu.sync_copy(zero_v, accum.at[pl.ds(sid * RPS, RPS)])
    pltpu.sync_copy(hp_hbm.at[pl.ds(sid * RPS, RPS)],
                    hp_s.at[pl.ds(sid * RPS, RPS)])
    wbase = (cid * NSUB + sid) * NCHUNKS
    pltpu.sync_copy(row_hbm.at[pl.ds(wbase, NCHUNKS)], rv_all)
    pltpu.sync_copy(col_hbm.at[pl.ds(wbase, NCHUNKS)], cv_all)
    plsc.subcore_barrier()

    @pl.loop(0, NCHUNKS, step=NBUF)
    def _(k):
        for j in range(NBUF):
            pltpu.async_copy(hp_s.at[rv_all.at[k + j]], msg_v.at[j], sem_g)
        for j in range(NBUF):
            pltpu.make_async_copy(hp_s.at[rv_all.at[k]], msg_v.at[0],
                                  sem_g).wait()
        for j in range(NBUF):
            pltpu.async_copy(msg_v.at[j], accum.at[cv_all.at[k + j]], sem_s,
                             add=True)
        for j in range(NBUF):
            pltpu.make_async_copy(msg_v.at[0], accum.at[cv_all.at[k]],
                                  sem_s).wait()

    plsc.subcore_barrier()
    pltpu.sync_copy(
        accum.at[pl.ds(sid * RPS, RPS)],
        out_hbm.at[pl.ds(cid * NPAD + sid * RPS, RPS)],
    )


def _tc_mm1(x_ref, w1t_ref, b1_ref, h1_ref):
    h1_ref[...] = lax.dot_general(
        x_ref[...], w1t_ref[...], (((1,), (0,)), ((), ())),
        preferred_element_type=jnp.float32,
    ) + b1_ref[...]


def _tc_scale1(deg_ref, h1_ref, hp_ref, dis_ref):
    deg = deg_ref[0, :N, :] + deg_ref[1, :N, :] + 1.0
    dis = lax.rsqrt(deg)
    hp_ref[:N, :] = dis * h1_ref[...]
    hp_ref[N:, :] = jnp.zeros((NPAD - N, 16), jnp.float32)
    dis_ref[...] = dis


def _tc_mid(p_ref, hp1_ref, dis_ref, w2t_ref, b2_ref, hp2_ref):
    dis = dis_ref[...]
    s = p_ref[0, :N, :] + p_ref[1, :N, :] + hp1_ref[:N, :]
    out1 = jnp.maximum(dis * s, 0.0)
    h2 = lax.dot_general(
        out1, w2t_ref[...], (((1,), (0,)), ((), ())),
        preferred_element_type=jnp.float32,
    ) + b2_ref[...]
    hp2_ref[:N, :] = dis * h2
    hp2_ref[N:, :] = jnp.zeros((NPAD - N, 16), jnp.float32)


def _tc_post(q_ref, hp2_ref, dis_ref, o_ref):
    o = dis_ref[...] * (q_ref[0, :N, :] + q_ref[1, :N, :] + hp2_ref[:N, :])
    m = jnp.max(o, axis=1, keepdims=True)
    lse = jnp.log(jnp.sum(jnp.exp(o - m), axis=1, keepdims=True)) + m
    o_ref[...] = o - lse


_mm1_call = pl.pallas_call(
    _tc_mm1,
    out_shape=jax.ShapeDtypeStruct((N, 16), jnp.float32),
)

_scale1_call = pl.pallas_call(
    _tc_scale1,
    out_shape=(
        jax.ShapeDtypeStruct((NPAD, 16), jnp.float32),
        jax.ShapeDtypeStruct((N, 16), jnp.float32),
    ),
)

_mid_call = pl.pallas_call(
    _tc_mid,
    out_shape=jax.ShapeDtypeStruct((NPAD, 16), jnp.float32),
)

_post_call = pl.pallas_call(
    _tc_post,
    out_shape=jax.ShapeDtypeStruct((N, 16), jnp.float32),
)


def kernel(x, edge_index, W1, b1, W2, b2):
    pad = jnp.full((2, EPAD - E), N, jnp.int32)
    ei = jnp.concatenate([edge_index, pad], axis=1)
    row2d = ei[0].reshape(NWORK * NCHUNKS, CHUNK)
    col2d = ei[1].reshape(NWORK * NCHUNKS, CHUNK)

    degp = _deg_pass(row2d).reshape(NCORES, NPAD, 16)
    h1 = _mm1_call(x, W1.T, b1.reshape(1, NHID))
    hp1, dis = _scale1_call(degp, h1)
    p = _agg_pass(hp1, row2d, col2d).reshape(NCORES, NPAD, 16)
    hp2 = _mid_call(p, hp1, dis, W2.T, b2.reshape(1, NHID))
    q = _agg_pass(hp2, row2d, col2d).reshape(NCORES, NPAD, 16)
    return _post_call(q, hp2, dis)


# speedup vs baseline: 56.8205x; 1.0358x over previous
import functools

import jax
import jax.numpy as jnp
from jax import lax
from jax.experimental import pallas as pl
from jax.experimental.pallas import tpu as pltpu
from jax.experimental.pallas import tpu_sc as plsc

N = 10000
NFEAT = 128
NHID = 16
NPAD = 10240
E = 320000
NCORES = 2
NSUB = 16
NWORK = NCORES * NSUB
CHUNK = 128
EPW = 10240
NCHUNKS = EPW // CHUNK
EPAD = NWORK * EPW
RPS = NPAD // NSUB
GRP = 4
NBUF = 2 * GRP
NBATCH = NCHUNKS // GRP

_mesh = plsc.VectorSubcoreMesh(core_axis_name="c", subcore_axis_name="s")
_sc_params = pltpu.CompilerParams(use_tc_tiling_on_sc=False)


def _zero_fill(buf, nrows):
    @pl.loop(0, nrows)
    def _(i):
        buf[i, :] = jnp.zeros((16,), jnp.float32)


@functools.partial(
    pl.kernel,
    out_type=jax.ShapeDtypeStruct((NCORES * NPAD, 16), jnp.float32),
    mesh=_mesh,
    scratch_types=[
        pltpu.VMEM((CHUNK, 16), jnp.float32),
        pltpu.VMEM((NCHUNKS, CHUNK), jnp.int32),
        pltpu.VMEM((RPS, 16), jnp.float32),
        pltpu.VMEM_SHARED((NPAD, 16), jnp.float32),
        pltpu.SemaphoreType.DMA,
    ],
    compiler_params=_sc_params,
)
def _deg_pass(row_hbm, out_hbm, ones_v, rv_all, zero_v, accum, sem):
    cid = lax.axis_index("c")
    sid = lax.axis_index("s")

    @pl.loop(0, CHUNK)
    def _(i):
        ones_v[i, :] = jnp.ones((16,), jnp.float32)

    _zero_fill(zero_v, RPS)
    pltpu.sync_copy(zero_v, accum.at[pl.ds(sid * RPS, RPS)])
    wbase = (cid * NSUB + sid) * NCHUNKS
    pltpu.sync_copy(row_hbm.at[pl.ds(wbase, NCHUNKS)], rv_all)
    plsc.subcore_barrier()

    @pl.loop(0, NCHUNKS)
    def _(k):
        pltpu.async_copy(ones_v, accum.at[rv_all.at[k]], sem, add=True)

    @pl.loop(0, NCHUNKS)
    def _(k):
        pltpu.make_async_copy(ones_v, accum.at[rv_all.at[0]], sem).wait()

    plsc.subcore_barrier()
    pltpu.sync_copy(
        accum.at[pl.ds(sid * RPS, RPS)],
        out_hbm.at[pl.ds(cid * NPAD + sid * RPS, RPS)],
    )


@functools.partial(
    pl.kernel,
    out_type=jax.ShapeDtypeStruct((NCORES * NPAD, 16), jnp.float32),
    mesh=_mesh,
    scratch_types=[
        pltpu.VMEM((NCHUNKS, CHUNK), jnp.int32),
        pltpu.VMEM((NCHUNKS, CHUNK), jnp.int32),
        pltpu.VMEM((NBUF, CHUNK, 16), jnp.float32),
        pltpu.VMEM((RPS, 16), jnp.float32),
        pltpu.VMEM_SHARED((NPAD, 16), jnp.float32),
        pltpu.VMEM_SHARED((NPAD, 16), jnp.float32),
        pltpu.SemaphoreType.DMA,
        pltpu.SemaphoreType.DMA,
    ],
    compiler_params=_sc_params,
)
def _agg_pass(hp_hbm, row_hbm, col_hbm, out_hbm, rv_all, cv_all, msg_v,
              zero_v, hp_s, accum, sem_g, sem_s):
    cid = lax.axis_index("c")
    sid = lax.axis_index("s")

    _zero_fill(zero_v, RPS)
    pltpu.sync_copy(zero_v, accum.at[pl.ds(sid * RPS, RPS)])
    pltpu.sync_copy(hp_hbm.at[pl.ds(sid * RPS, RPS)],
                    hp_s.at[pl.ds(sid * RPS, RPS)])
    wbase = (cid * NSUB + sid) * NCHUNKS
    pltpu.sync_copy(row_hbm.at[pl.ds(wbase, NCHUNKS)], rv_all)
    pltpu.sync_copy(col_hbm.at[pl.ds(wbase, NCHUNKS)], cv_all)
    plsc.subcore_barrier()

    for j in range(GRP):
        pltpu.async_copy(hp_s.at[rv_all.at[j]], msg_v.at[j], sem_g)
    for j in range(GRP):
        pltpu.make_async_copy(hp_s.at[rv_all.at[0]], msg_v.at[0],
                              sem_g).wait()
    for j in range(GRP):
        pltpu.async_copy(msg_v.at[j], accum.at[cv_all.at[j]], sem_s,
                         add=True)
    for j in range(GRP):
        pltpu.async_copy(hp_s.at[rv_all.at[GRP + j]], msg_v.at[GRP + j],
                         sem_g)

    @pl.loop(1, NBATCH)
    def _(k):
        half = (k % 2) * GRP
        nhalf = ((k + 1) % 2) * GRP
        base = k * GRP
        nbase = ((k + 1) % NBATCH) * GRP
        for j in range(GRP):
            pltpu.make_async_copy(hp_s.at[rv_all.at[0]], msg_v.at[0],
                                  sem_g).wait()
        for j in range(GRP):
            pltpu.make_async_copy(msg_v.at[0], accum.at[cv_all.at[0]],
                                  sem_s).wait()
        for j in range(GRP):
            pltpu.async_copy(msg_v.at[half + j],
                             accum.at[cv_all.at[base + j]], sem_s, add=True)
        for j in range(GRP):
            pltpu.async_copy(hp_s.at[rv_all.at[nbase + j]],
                             msg_v.at[nhalf + j], sem_g)

    for j in range(GRP):
        pltpu.make_async_copy(hp_s.at[rv_all.at[0]], msg_v.at[0],
                              sem_g).wait()
    for j in range(GRP):
        pltpu.make_async_copy(msg_v.at[0], accum.at[cv_all.at[0]],
                              sem_s).wait()

    plsc.subcore_barrier()
    pltpu.sync_copy(
        accum.at[pl.ds(sid * RPS, RPS)],
        out_hbm.at[pl.ds(cid * NPAD + sid * RPS, RPS)],
    )


def _tc_mm1(x_ref, w1t_ref, b1_ref, h1_ref):
    h1_ref[...] = lax.dot_general(
        x_ref[...], w1t_ref[...], (((1,), (0,)), ((), ())),
        preferred_element_type=jnp.float32,
    ) + b1_ref[...]


def _tc_scale1(deg_ref, h1_ref, hp_ref, dis_ref):
    deg = deg_ref[0, :N, :] + deg_ref[1, :N, :] + 1.0
    dis = lax.rsqrt(deg)
    hp_ref[:N, :] = dis * h1_ref[...]
    hp_ref[N:, :] = jnp.zeros((NPAD - N, 16), jnp.float32)
    dis_ref[...] = dis


def _tc_mid(p_ref, hp1_ref, dis_ref, w2t_ref, b2_ref, hp2_ref):
    dis = dis_ref[...]
    s = p_ref[0, :N, :] + p_ref[1, :N, :] + hp1_ref[:N, :]
    out1 = jnp.maximum(dis * s, 0.0)
    h2 = lax.dot_general(
        out1, w2t_ref[...], (((1,), (0,)), ((), ())),
        preferred_element_type=jnp.float32,
    ) + b2_ref[...]
    hp2_ref[:N, :] = dis * h2
    hp2_ref[N:, :] = jnp.zeros((NPAD - N, 16), jnp.float32)


def _tc_post(q_ref, hp2_ref, dis_ref, o_ref):
    o = dis_ref[...] * (q_ref[0, :N, :] + q_ref[1, :N, :] + hp2_ref[:N, :])
    m = jnp.max(o, axis=1, keepdims=True)
    lse = jnp.log(jnp.sum(jnp.exp(o - m), axis=1, keepdims=True)) + m
    o_ref[...] = o - lse


_mm1_call = pl.pallas_call(
    _tc_mm1,
    out_shape=jax.ShapeDtypeStruct((N, 16), jnp.float32),
)

_scale1_call = pl.pallas_call(
    _tc_scale1,
    out_shape=(
        jax.ShapeDtypeStruct((NPAD, 16), jnp.float32),
        jax.ShapeDtypeStruct((N, 16), jnp.float32),
    ),
)

_mid_call = pl.pallas_call(
    _tc_mid,
    out_shape=jax.ShapeDtypeStruct((NPAD, 16), jnp.float32),
)

_post_call = pl.pallas_call(
    _tc_post,
    out_shape=jax.ShapeDtypeStruct((N, 16), jnp.float32),
)


def kernel(x, edge_index, W1, b1, W2, b2):
    pad = jnp.full((2, EPAD - E), N, jnp.int32)
    ei = jnp.concatenate([edge_index, pad], axis=1)
    row2d = ei[0].reshape(NWORK * NCHUNKS, CHUNK)
    col2d = ei[1].reshape(NWORK * NCHUNKS, CHUNK)

    degp = _deg_pass(row2d).reshape(NCORES, NPAD, 16)
    h1 = _mm1_call(x, W1.T, b1.reshape(1, NHID))
    hp1, dis = _scale1_call(degp, h1)
    p = _agg_pass(hp1, row2d, col2d).reshape(NCORES, NPAD, 16)
    hp2 = _mid_call(p, hp1, dis, W2.T, b2.reshape(1, NHID))
    q = _agg_pass(hp2, row2d, col2d).reshape(NCORES, NPAD, 16)
    return _post_call(q, hp2, dis)
